# Initial kernel scaffold; baseline (speedup 1.0000x reference)
import jax
import jax.numpy as jnp
from jax.experimental import pallas as pl


def kernel(coords, samples, sigma, w_lift, b_lift, w_proj, b_proj, w0, b0, w1, b1, g00_w1, g00_b1, g00_w2, g00_b2, g01_w1, g01_b1, g01_w2, g01_b2, g10_w1, g10_b1, g10_w2, g10_b2, g11_w1, g11_b1, g11_w2, g11_b2):
    raise NotImplementedError("write your pallas kernel here")


# speedup vs baseline: 4.4767x; 4.4767x over previous
import functools

import jax
import jax.numpy as jnp
from jax.experimental import pallas as pl
from jax.experimental.pallas import tpu as pltpu

BQ = 128
BJ = 128
LAT = 64


def _gno_body(jlo_ref, jhi_ref, xq_ref, ys_ref, f_ref, w1_ref, b1_ref,
              w2_ref, b2_ref, *rest, r2, nchunks, apply_relu, has_res):
    if has_res:
        res_ref, out_ref, acc_ref, cnt_ref = rest
    else:
        out_ref, acc_ref, cnt_ref = rest
        res_ref = None
    b = pl.program_id(0)
    j = pl.program_id(1)

    @pl.when(j == 0)
    def _init():
        acc_ref[...] = jnp.zeros_like(acc_ref)
        cnt_ref[...] = jnp.zeros_like(cnt_ref)

    active = jnp.logical_and(j >= jlo_ref[b], j <= jhi_ref[b])

    @pl.when(active)
    def _compute():
        xq = xq_ref[...]
        ys = ys_ref[pl.ds(j * BJ, BJ), :]
        fb = f_ref[pl.ds(j * BJ, BJ), :]
        dx = xq[:, 0:1] - ys[:, 0][None, :]
        dy = xq[:, 1:2] - ys[:, 1][None, :]
        d2 = dx * dx + dy * dy
        m = (d2 <= r2).astype(jnp.float32)
        u = jnp.dot(ys, w1_ref[0:2, :], preferred_element_type=jnp.float32)
        v = jnp.dot(xq, w1_ref[2:4, :], preferred_element_type=jnp.float32)
        v = v + b1_ref[...]
        pre = v[:, None, :] + u[None, :, :]
        h = jax.nn.gelu(pre)
        k = jnp.dot(h.reshape(BQ * BJ, LAT), w2_ref[...],
                    preferred_element_type=jnp.float32)
        k = k.reshape(BQ, BJ, LAT) + b2_ref[...][None, :, :]
        vals = k * fb[None, :, :]
        acc_ref[...] += jnp.sum(vals * m[:, :, None], axis=1)
        cnt_ref[...] += jnp.sum(m, axis=1, keepdims=True)

    @pl.when(j == nchunks - 1)
    def _finalize():
        out = acc_ref[...] / jnp.maximum(cnt_ref[...], 1.0)
        if res_ref is not None:
            out = out + res_ref[...]
        if apply_relu:
            out = jnp.maximum(out, 0.0)
        out_ref[...] = out


def _gno(xq, ys, f, w1, b1, w2, b2, radius, res=None, apply_relu=False):
    nx = xq.shape[0]
    ny = ys.shape[0]
    nblocks = nx // BQ
    nchunks = ny // BJ
    qx = xq[:, 0].reshape(nblocks, BQ)
    qmin, qmax = qx[:, 0], qx[:, -1]
    cx = ys[:, 0].reshape(nchunks, BJ)
    cmin, cmax = cx[:, 0], cx[:, -1]
    jlo = jnp.searchsorted(cmax, qmin - radius, side="left").astype(jnp.int32)
    jhi = (jnp.searchsorted(cmin, qmax + radius, side="right") - 1).astype(jnp.int32)

    body = functools.partial(
        _gno_body, r2=radius * radius, nchunks=nchunks,
        apply_relu=apply_relu, has_res=res is not None)

    in_specs = [
        pl.BlockSpec(memory_space=pltpu.SMEM),
        pl.BlockSpec(memory_space=pltpu.SMEM),
        pl.BlockSpec((BQ, 2), lambda b, j: (b, 0)),
        pl.BlockSpec((ny, 2), lambda b, j: (0, 0)),
        pl.BlockSpec((ny, LAT), lambda b, j: (0, 0)),
        pl.BlockSpec((4, LAT), lambda b, j: (0, 0)),
        pl.BlockSpec((1, LAT), lambda b, j: (0, 0)),
        pl.BlockSpec((LAT, LAT), lambda b, j: (0, 0)),
        pl.BlockSpec((1, LAT), lambda b, j: (0, 0)),
    ]
    args = [jlo, jhi, xq, ys, f, w1, b1.reshape(1, LAT), w2, b2.reshape(1, LAT)]
    if res is not None:
        in_specs.append(pl.BlockSpec((BQ, LAT), lambda b, j: (b, 0)))
        args.append(res)

    return pl.pallas_call(
        body,
        grid=(nblocks, nchunks),
        in_specs=in_specs,
        out_specs=pl.BlockSpec((BQ, LAT), lambda b, j: (b, 0)),
        out_shape=jax.ShapeDtypeStruct((nx, LAT), jnp.float32),
        scratch_shapes=[
            pltpu.VMEM((BQ, LAT), jnp.float32),
            pltpu.VMEM((BQ, 1), jnp.float32),
        ],
        compiler_params=pltpu.CompilerParams(
            dimension_semantics=("arbitrary", "arbitrary")),
    )(*args)


def _dense_body(x_ref, w_ref, b_ref, *rest, apply_relu, has_res):
    if has_res:
        res_ref, out_ref = rest
    else:
        (out_ref,) = rest
        res_ref = None
    y = jnp.dot(x_ref[...], w_ref[...], preferred_element_type=jnp.float32)
    y = y + b_ref[...]
    if res_ref is not None:
        y = y + res_ref[...]
    if apply_relu:
        y = jnp.maximum(y, 0.0)
    out_ref[...] = y


def _dense(x, w, b, res=None, apply_relu=False):
    n, di = x.shape
    do = w.shape[1]
    body = functools.partial(_dense_body, apply_relu=apply_relu,
                             has_res=res is not None)
    args = [x, w, b.reshape(1, do)]
    if res is not None:
        args.append(res)
    return pl.pallas_call(
        body,
        out_shape=jax.ShapeDtypeStruct((n, do), jnp.float32),
    )(*args)


def kernel(coords, samples, sigma, w_lift, b_lift, w_proj, b_proj, w0, b0,
           w1, b1, g00_w1, g00_b1, g00_w2, g00_b2, g01_w1, g01_b1, g01_w2,
           g01_b2, g10_w1, g10_b1, g10_w2, g10_b2, g11_w1, g11_b1, g11_w2,
           g11_b2):
    del sigma
    r0, r1 = 0.03, 0.06
    c0 = coords.T
    c1 = coords[:, ::2].T
    p0 = jnp.argsort(c0[:, 0])
    p1 = jnp.argsort(c1[:, 0])
    s0 = c0[p0]
    s1 = c1[p1]
    inv_p0 = jnp.argsort(p0)

    samples0 = samples[p0]
    pad = jnp.zeros((8 - samples0.shape[1], LAT), jnp.float32)
    up0 = _dense(jnp.pad(samples0, ((0, 0), (0, 8 - samples0.shape[1]))),
                 jnp.concatenate([w_lift, pad], axis=0), b_lift)

    a01 = _gno(s1, s0, up0, g01_w1, g01_b1, g01_w2, g01_b2, r1)
    g00 = _gno(s0, s0, up0, g00_w1, g00_b1, g00_w2, g00_b2, r0)
    cterm = _dense(up0, w0, b0, res=g00)

    vu1 = None
    for _ in range(3):
        vd1 = jnp.maximum(a01 + vu1, 0.0) if vu1 is not None else jnp.maximum(a01, 0.0)
        g11 = _gno(s1, s1, vd1, g11_w1, g11_b1, g11_w2, g11_b2, r1)
        vu1 = _dense(vd1, w1, b1, res=g11, apply_relu=True)

    v0 = _gno(s0, s1, vu1, g10_w1, g10_b1, g10_w2, g10_b2, r1,
              res=cterm, apply_relu=True)

    wp = jnp.pad(w_proj, ((0, 0), (0, 128 - w_proj.shape[1])))
    bp = jnp.pad(b_proj, ((0, 128 - b_proj.shape[0]),))
    out = _dense(v0, wp, bp)[:, : w_proj.shape[1]]
    return out[inv_p0]
